# Initial kernel scaffold; baseline (speedup 1.0000x reference)
import jax
import jax.numpy as jnp
from jax.experimental import pallas as pl


def kernel(x, edge_index, batch, W1_rel, b1, W1_root, W2_rel, b2, W2_root, W3_rel, b3, W3_root, W_lin, b_lin):
    raise NotImplementedError("write your pallas kernel here")


# speedup vs baseline: 6.9295x; 6.9295x over previous
import functools

import jax
import jax.numpy as jnp
from jax import lax
from jax.experimental import pallas as pl
from jax.experimental.pallas import tpu as pltpu
from jax.experimental.pallas import tpu_sc as plsc

N = 10000
E = 320000
D = 128
G = 64

NC = 2
NS = 16
NW = NC * NS

EDGE_CHUNK = 80
E_PER_TILE = E // NW
CHUNKS_PER_TILE = E_PER_TILE // EDGE_CHUNK
N_CHUNKS = N // EDGE_CHUNK
ROWS_T = 624
ROWS_TAIL = N - NS * ROWS_T


def _seg_sum_body(h_hbm, srcr_hbm, dstr_hbm, zeros_hbm, out_hbm,
                  src_v, dst_v, rows_v, acc, gsem):
  c = lax.axis_index("c")
  s = lax.axis_index("s")
  w = c * NS + s

  zbase = pl.multiple_of(s * ROWS_T, 8)
  pltpu.sync_copy(zeros_hbm.at[pl.ds(zbase, ROWS_T)],
                  acc.at[pl.ds(zbase, ROWS_T)])

  @pl.when(s == NS - 1)
  def _():
    pltpu.sync_copy(zeros_hbm.at[pl.ds(NS * ROWS_T, ROWS_TAIL)],
                    acc.at[pl.ds(NS * ROWS_T, ROWS_TAIL)])

  pltpu.sync_copy(srcr_hbm.at[w], src_v)
  pltpu.sync_copy(dstr_hbm.at[w], dst_v)
  plsc.subcore_barrier()

  def chunk(g, carry):
    pltpu.async_copy(h_hbm.at[src_v.at[g]], rows_v, gsem).wait()
    pltpu.sync_copy(rows_v, acc.at[dst_v.at[g]], add=True)
    return carry

  lax.fori_loop(0, CHUNKS_PER_TILE, chunk, 0, unroll=False)

  plsc.subcore_barrier()
  pltpu.sync_copy(acc.at[pl.ds(zbase, ROWS_T)],
                  out_hbm.at[c, pl.ds(zbase, ROWS_T)])

  @pl.when(s == NS - 1)
  def _():
    pltpu.sync_copy(acc.at[pl.ds(NS * ROWS_T, ROWS_TAIL)],
                    out_hbm.at[c, pl.ds(NS * ROWS_T, ROWS_TAIL)])


def _seg_sum_sc(h, src_r, dst_r, zeros):
  mesh = plsc.VectorSubcoreMesh(core_axis_name="c", subcore_axis_name="s")
  return pl.kernel(
      _seg_sum_body,
      out_type=jax.ShapeDtypeStruct((NC, N, D), jnp.float32),
      mesh=mesh,
      scratch_types=[
          pltpu.VMEM((CHUNKS_PER_TILE, EDGE_CHUNK), jnp.int32),
          pltpu.VMEM((CHUNKS_PER_TILE, EDGE_CHUNK), jnp.int32),
          pltpu.VMEM((EDGE_CHUNK, D), jnp.float32),
          pltpu.VMEM_SHARED((N, D), jnp.float32),
          pltpu.SemaphoreType.DMA,
      ],
  )(h, src_r, dst_r, zeros)


def _pool_body(h_hbm, batchr_hbm, ones_hbm, zeros_hbm, sums_hbm, cnts_hbm,
               batch_v, rows_v, ones_v, sums, cnts, gsem):
  c = lax.axis_index("c")
  s = lax.axis_index("s")
  w = c * NS + s

  @pl.when(s == 0)
  def _():
    pltpu.sync_copy(zeros_hbm.at[pl.ds(0, G)], sums)
    pltpu.sync_copy(zeros_hbm.at[pl.ds(G, G)], cnts)

  pltpu.sync_copy(ones_hbm, ones_v)
  pltpu.sync_copy(batchr_hbm, batch_v)
  plsc.subcore_barrier()

  def chunk(k, carry):
    g = w + k * NW
    @pl.when(g < N_CHUNKS)
    def _():
      base = pl.multiple_of(g * EDGE_CHUNK, 8)
      pltpu.async_copy(h_hbm.at[pl.ds(base, EDGE_CHUNK)], rows_v,
                       gsem).wait()
      pltpu.sync_copy(rows_v, sums.at[batch_v.at[g]], add=True)
      pltpu.sync_copy(ones_v, cnts.at[batch_v.at[g]], add=True)
    return carry

  lax.fori_loop(0, (N_CHUNKS + NW - 1) // NW, chunk, 0, unroll=False)

  plsc.subcore_barrier()

  @pl.when(s == 0)
  def _():
    pltpu.sync_copy(sums, sums_hbm.at[c])
    pltpu.sync_copy(cnts, cnts_hbm.at[c])


def _pool_sc(h, batch_r, ones, zeros):
  mesh = plsc.VectorSubcoreMesh(core_axis_name="c", subcore_axis_name="s")
  return pl.kernel(
      _pool_body,
      out_type=(jax.ShapeDtypeStruct((NC, G, D), jnp.float32),
                jax.ShapeDtypeStruct((NC, G, D), jnp.float32)),
      mesh=mesh,
      scratch_types=[
          pltpu.VMEM((N_CHUNKS, EDGE_CHUNK), jnp.int32),
          pltpu.VMEM((EDGE_CHUNK, D), jnp.float32),
          pltpu.VMEM((EDGE_CHUNK, D), jnp.float32),
          pltpu.VMEM_SHARED((G, D), jnp.float32),
          pltpu.VMEM_SHARED((G, D), jnp.float32),
          pltpu.SemaphoreType.DMA,
      ],
  )(h, batch_r, ones, zeros)


BN = 1000


def _dense_body(relu, p0_ref, p1_ref, h_ref, wr_ref, wroot_ref, b_ref,
                out_ref):
  agg = p0_ref[...] + p1_ref[...]
  r = (jnp.dot(agg, wr_ref[...], preferred_element_type=jnp.float32)
       + jnp.dot(h_ref[...], wroot_ref[...],
                 preferred_element_type=jnp.float32)
       + b_ref[...])
  out_ref[...] = jnp.maximum(r, 0.0) if relu else r


def _dense_layer(parts, h, wr, wroot, b, relu):
  p0 = parts[0]
  p1 = parts[1]
  b2 = b.reshape(1, D)
  grid = (N // BN,)
  return pl.pallas_call(
      functools.partial(_dense_body, relu),
      grid=grid,
      in_specs=[
          pl.BlockSpec((BN, D), lambda i: (i, 0)),
          pl.BlockSpec((BN, D), lambda i: (i, 0)),
          pl.BlockSpec((BN, D), lambda i: (i, 0)),
          pl.BlockSpec((D, D), lambda i: (0, 0)),
          pl.BlockSpec((D, D), lambda i: (0, 0)),
          pl.BlockSpec((1, D), lambda i: (0, 0)),
      ],
      out_specs=pl.BlockSpec((BN, D), lambda i: (i, 0)),
      out_shape=jax.ShapeDtypeStruct((N, D), jnp.float32),
  )(p0, p1, h, wr, wroot, b2)


def _head_body(s0_ref, s1_ref, c0_ref, c1_ref, wl_ref, bl_ref, out_ref):
  sums = s0_ref[...] + s1_ref[...]
  cnts = c0_ref[...] + c1_ref[...]
  pooled = sums / jnp.maximum(cnts, 1.0)
  out_ref[...] = (jnp.dot(pooled, wl_ref[...],
                          preferred_element_type=jnp.float32)
                  + bl_ref[...])


def _head(sums, cnts, w_lin, b_lin):
  return pl.pallas_call(
      _head_body,
      out_shape=jax.ShapeDtypeStruct((G, 2), jnp.float32),
  )(sums[0], sums[1], cnts[0], cnts[1], w_lin, b_lin.reshape(1, 2))


def kernel(x, edge_index, batch, W1_rel, b1, W1_root, W2_rel, b2, W2_root,
           W3_rel, b3, W3_root, W_lin, b_lin):
  src_r = edge_index[0].reshape(NW, CHUNKS_PER_TILE, EDGE_CHUNK)
  dst_r = edge_index[1].reshape(NW, CHUNKS_PER_TILE, EDGE_CHUNK)
  batch_r = batch.reshape(N_CHUNKS, EDGE_CHUNK)
  zeros_n = jnp.zeros((N, D), jnp.float32)
  zeros_g = jnp.zeros((2 * G, D), jnp.float32)
  ones = jnp.ones((EDGE_CHUNK, D), jnp.float32)

  h = x
  p = _seg_sum_sc(h, src_r, dst_r, zeros_n)
  h = _dense_layer(p, h, W1_rel, W1_root, b1, relu=True)
  p = _seg_sum_sc(h, src_r, dst_r, zeros_n)
  h = _dense_layer(p, h, W2_rel, W2_root, b2, relu=True)
  p = _seg_sum_sc(h, src_r, dst_r, zeros_n)
  h = _dense_layer(p, h, W3_rel, W3_root, b3, relu=False)

  sums, cnts = _pool_sc(h, batch_r, ones, zeros_g)
  return _head(sums, cnts, W_lin, b_lin)


# speedup vs baseline: 7.9299x; 1.1444x over previous
import functools

import jax
import jax.numpy as jnp
from jax import lax
from jax.experimental import pallas as pl
from jax.experimental.pallas import tpu as pltpu
from jax.experimental.pallas import tpu_sc as plsc

N = 10000
E = 320000
D = 128
G = 64

NC = 2
NS = 16
NW = NC * NS

EDGE_CHUNK = 125
E_PER_TILE = E // NW
CHUNKS_PER_TILE = E_PER_TILE // EDGE_CHUNK
POOL_CHUNK = 80
N_CHUNKS = N // POOL_CHUNK
ROWS_T = 624
ROWS_TAIL = N - NS * ROWS_T


def _seg_sum_body(h_hbm, srcr_hbm, dstr_hbm, zeros_hbm, out_hbm,
                  src_v, dst_v, rows0_v, acc, sem0):
  c = lax.axis_index("c")
  s = lax.axis_index("s")
  w = c * NS + s

  zbase = pl.multiple_of(s * ROWS_T, 8)
  pltpu.sync_copy(zeros_hbm.at[pl.ds(zbase, ROWS_T)],
                  acc.at[pl.ds(zbase, ROWS_T)])

  @pl.when(s == NS - 1)
  def _():
    pltpu.sync_copy(zeros_hbm.at[pl.ds(NS * ROWS_T, ROWS_TAIL)],
                    acc.at[pl.ds(NS * ROWS_T, ROWS_TAIL)])

  pltpu.sync_copy(srcr_hbm.at[w], src_v)
  pltpu.sync_copy(dstr_hbm.at[w], dst_v)
  plsc.subcore_barrier()

  def chunk(g, carry):
    pltpu.async_copy(h_hbm.at[src_v.at[g]], rows0_v, sem0).wait()
    pltpu.sync_copy(rows0_v, acc.at[dst_v.at[g]], add=True)
    return carry

  lax.fori_loop(0, CHUNKS_PER_TILE, chunk, 0, unroll=False)

  plsc.subcore_barrier()
  pltpu.sync_copy(acc.at[pl.ds(zbase, ROWS_T)],
                  out_hbm.at[c, pl.ds(zbase, ROWS_T)])

  @pl.when(s == NS - 1)
  def _():
    pltpu.sync_copy(acc.at[pl.ds(NS * ROWS_T, ROWS_TAIL)],
                    out_hbm.at[c, pl.ds(NS * ROWS_T, ROWS_TAIL)])


def _seg_sum_sc(h, src_r, dst_r, zeros):
  mesh = plsc.VectorSubcoreMesh(core_axis_name="c", subcore_axis_name="s")
  return pl.kernel(
      _seg_sum_body,
      out_type=jax.ShapeDtypeStruct((NC, N, D), jnp.float32),
      mesh=mesh,
      scratch_types=[
          pltpu.VMEM((CHUNKS_PER_TILE, EDGE_CHUNK), jnp.int32),
          pltpu.VMEM((CHUNKS_PER_TILE, EDGE_CHUNK), jnp.int32),
          pltpu.VMEM((EDGE_CHUNK, D), jnp.float32),
          pltpu.VMEM_SHARED((N, D), jnp.float32),
          pltpu.SemaphoreType.DMA,
      ],
  )(h, src_r, dst_r, zeros)


def _pool_body(h_hbm, batchr_hbm, ones_hbm, zeros_hbm, sums_hbm, cnts_hbm,
               batch_v, rows_v, ones_v, sums, cnts, gsem):
  c = lax.axis_index("c")
  s = lax.axis_index("s")
  w = c * NS + s

  @pl.when(s == 0)
  def _():
    pltpu.sync_copy(zeros_hbm.at[pl.ds(0, G)], sums)
    pltpu.sync_copy(zeros_hbm.at[pl.ds(G, G)], cnts)

  pltpu.sync_copy(ones_hbm, ones_v)
  pltpu.sync_copy(batchr_hbm, batch_v)
  plsc.subcore_barrier()

  def chunk(k, carry):
    g = w + k * NW
    @pl.when(g < N_CHUNKS)
    def _():
      base = pl.multiple_of(g * POOL_CHUNK, 8)
      pltpu.async_copy(h_hbm.at[pl.ds(base, POOL_CHUNK)], rows_v,
                       gsem).wait()
      pltpu.sync_copy(rows_v, sums.at[batch_v.at[g]], add=True)
      pltpu.sync_copy(ones_v, cnts.at[batch_v.at[g]], add=True)
    return carry

  lax.fori_loop(0, (N_CHUNKS + NW - 1) // NW, chunk, 0, unroll=False)

  plsc.subcore_barrier()

  @pl.when(s == 0)
  def _():
    pltpu.sync_copy(sums, sums_hbm.at[c])
    pltpu.sync_copy(cnts, cnts_hbm.at[c])


def _pool_sc(h, batch_r, ones, zeros):
  mesh = plsc.VectorSubcoreMesh(core_axis_name="c", subcore_axis_name="s")
  return pl.kernel(
      _pool_body,
      out_type=(jax.ShapeDtypeStruct((NC, G, D), jnp.float32),
                jax.ShapeDtypeStruct((NC, G, D), jnp.float32)),
      mesh=mesh,
      scratch_types=[
          pltpu.VMEM((N_CHUNKS, POOL_CHUNK), jnp.int32),
          pltpu.VMEM((POOL_CHUNK, D), jnp.float32),
          pltpu.VMEM((POOL_CHUNK, D), jnp.float32),
          pltpu.VMEM_SHARED((G, D), jnp.float32),
          pltpu.VMEM_SHARED((G, D), jnp.float32),
          pltpu.SemaphoreType.DMA,
      ],
  )(h, batch_r, ones, zeros)


BN = 1000


def _dense_body(relu, p0_ref, p1_ref, h_ref, wr_ref, wroot_ref, b_ref,
                out_ref):
  agg = p0_ref[...] + p1_ref[...]
  r = (jnp.dot(agg, wr_ref[...], preferred_element_type=jnp.float32)
       + jnp.dot(h_ref[...], wroot_ref[...],
                 preferred_element_type=jnp.float32)
       + b_ref[...])
  out_ref[...] = jnp.maximum(r, 0.0) if relu else r


def _dense_layer(parts, h, wr, wroot, b, relu):
  p0 = parts[0]
  p1 = parts[1]
  b2 = b.reshape(1, D)
  grid = (N // BN,)
  return pl.pallas_call(
      functools.partial(_dense_body, relu),
      grid=grid,
      in_specs=[
          pl.BlockSpec((BN, D), lambda i: (i, 0)),
          pl.BlockSpec((BN, D), lambda i: (i, 0)),
          pl.BlockSpec((BN, D), lambda i: (i, 0)),
          pl.BlockSpec((D, D), lambda i: (0, 0)),
          pl.BlockSpec((D, D), lambda i: (0, 0)),
          pl.BlockSpec((1, D), lambda i: (0, 0)),
      ],
      out_specs=pl.BlockSpec((BN, D), lambda i: (i, 0)),
      out_shape=jax.ShapeDtypeStruct((N, D), jnp.float32),
  )(p0, p1, h, wr, wroot, b2)


def _head_body(s0_ref, s1_ref, c0_ref, c1_ref, wl_ref, bl_ref, out_ref):
  sums = s0_ref[...] + s1_ref[...]
  cnts = c0_ref[...] + c1_ref[...]
  pooled = sums / jnp.maximum(cnts, 1.0)
  out_ref[...] = (jnp.dot(pooled, wl_ref[...],
                          preferred_element_type=jnp.float32)
                  + bl_ref[...])


def _head(sums, cnts, w_lin, b_lin):
  return pl.pallas_call(
      _head_body,
      out_shape=jax.ShapeDtypeStruct((G, 2), jnp.float32),
  )(sums[0], sums[1], cnts[0], cnts[1], w_lin, b_lin.reshape(1, 2))


def kernel(x, edge_index, batch, W1_rel, b1, W1_root, W2_rel, b2, W2_root,
           W3_rel, b3, W3_root, W_lin, b_lin):
  src_r = edge_index[0].reshape(NW, CHUNKS_PER_TILE, EDGE_CHUNK)
  dst_r = edge_index[1].reshape(NW, CHUNKS_PER_TILE, EDGE_CHUNK)
  batch_r = batch.reshape(N_CHUNKS, POOL_CHUNK)
  zeros_n = jnp.zeros((N, D), jnp.float32)
  zeros_g = jnp.zeros((2 * G, D), jnp.float32)
  ones = jnp.ones((POOL_CHUNK, D), jnp.float32)

  h = x
  p = _seg_sum_sc(h, src_r, dst_r, zeros_n)
  h = _dense_layer(p, h, W1_rel, W1_root, b1, relu=True)
  p = _seg_sum_sc(h, src_r, dst_r, zeros_n)
  h = _dense_layer(p, h, W2_rel, W2_root, b2, relu=True)
  p = _seg_sum_sc(h, src_r, dst_r, zeros_n)
  h = _dense_layer(p, h, W3_rel, W3_root, b3, relu=False)

  sums, cnts = _pool_sc(h, batch_r, ones, zeros_g)
  return _head(sums, cnts, W_lin, b_lin)


# speedup vs baseline: 11.5651x; 1.4584x over previous
import functools

import jax
import jax.numpy as jnp
from jax import lax
from jax.experimental import pallas as pl
from jax.experimental.pallas import tpu as pltpu
from jax.experimental.pallas import tpu_sc as plsc

N = 10000
E = 320000
D = 128
G = 64

NC = 2
NS = 16
NW = NC * NS

EDGE_CHUNK = 125
KDEPTH = 4
E_PER_TILE = E // NW
CHUNKS_PER_TILE = E_PER_TILE // EDGE_CHUNK
GROUPS_PER_TILE = CHUNKS_PER_TILE // KDEPTH
POOL_CHUNK = 80
N_CHUNKS = N // POOL_CHUNK
ROWS_T = 624
ROWS_TAIL = N - NS * ROWS_T


def _seg_sum_body(h_hbm, srcr_hbm, dstr_hbm, zeros_hbm, out_hbm,
                  src_v, dst_v, rows_bufs, acc, sems):
  c = lax.axis_index("c")
  s = lax.axis_index("s")
  w = c * NS + s

  zbase = pl.multiple_of(s * ROWS_T, 8)
  pltpu.sync_copy(zeros_hbm.at[pl.ds(zbase, ROWS_T)],
                  acc.at[pl.ds(zbase, ROWS_T)])

  @pl.when(s == NS - 1)
  def _():
    pltpu.sync_copy(zeros_hbm.at[pl.ds(NS * ROWS_T, ROWS_TAIL)],
                    acc.at[pl.ds(NS * ROWS_T, ROWS_TAIL)])

  pltpu.sync_copy(srcr_hbm.at[w], src_v)
  pltpu.sync_copy(dstr_hbm.at[w], dst_v)
  plsc.subcore_barrier()

  def group(i, carry):
    g0 = i * KDEPTH
    for b in range(KDEPTH):
      pltpu.async_copy(h_hbm.at[src_v.at[g0 + b]], rows_bufs.at[b], sems[b])
    for b in range(KDEPTH):
      pltpu.make_async_copy(h_hbm.at[src_v.at[g0 + b]], rows_bufs.at[b],
                            sems[b]).wait()
      pltpu.sync_copy(rows_bufs.at[b], acc.at[dst_v.at[g0 + b]], add=True)
    return carry

  lax.fori_loop(0, GROUPS_PER_TILE, group, 0, unroll=False)

  plsc.subcore_barrier()
  pltpu.sync_copy(acc.at[pl.ds(zbase, ROWS_T)],
                  out_hbm.at[c, pl.ds(zbase, ROWS_T)])

  @pl.when(s == NS - 1)
  def _():
    pltpu.sync_copy(acc.at[pl.ds(NS * ROWS_T, ROWS_TAIL)],
                    out_hbm.at[c, pl.ds(NS * ROWS_T, ROWS_TAIL)])


def _seg_sum_sc(h, src_r, dst_r, zeros):
  mesh = plsc.VectorSubcoreMesh(core_axis_name="c", subcore_axis_name="s")
  return pl.kernel(
      _seg_sum_body,
      out_type=jax.ShapeDtypeStruct((NC, N, D), jnp.bfloat16),
      mesh=mesh,
      scratch_types=[
          pltpu.VMEM((CHUNKS_PER_TILE, EDGE_CHUNK), jnp.int32),
          pltpu.VMEM((CHUNKS_PER_TILE, EDGE_CHUNK), jnp.int32),
          pltpu.VMEM((KDEPTH, EDGE_CHUNK, D), jnp.bfloat16),
          pltpu.VMEM_SHARED((N, D), jnp.bfloat16),
          [pltpu.SemaphoreType.DMA] * KDEPTH,
      ],
      compiler_params=pltpu.CompilerParams(use_tc_tiling_on_sc=False),
  )(h, src_r, dst_r, zeros)


def _pool_body(h_hbm, batchr_hbm, ones_hbm, zeros_hbm, sums_hbm, cnts_hbm,
               batch_v, rows_v, ones_v, sums, cnts, gsem):
  c = lax.axis_index("c")
  s = lax.axis_index("s")
  w = c * NS + s

  @pl.when(s == 0)
  def _():
    pltpu.sync_copy(zeros_hbm.at[pl.ds(0, G)], sums)
    pltpu.sync_copy(zeros_hbm.at[pl.ds(G, G)], cnts)

  pltpu.sync_copy(ones_hbm, ones_v)
  pltpu.sync_copy(batchr_hbm, batch_v)
  plsc.subcore_barrier()

  def chunk(k, carry):
    g = w + k * NW
    @pl.when(g < N_CHUNKS)
    def _():
      base = pl.multiple_of(g * POOL_CHUNK, 8)
      pltpu.async_copy(h_hbm.at[pl.ds(base, POOL_CHUNK)], rows_v,
                       gsem).wait()
      pltpu.sync_copy(rows_v, sums.at[batch_v.at[g]], add=True)
      pltpu.sync_copy(ones_v, cnts.at[batch_v.at[g]], add=True)
    return carry

  lax.fori_loop(0, (N_CHUNKS + NW - 1) // NW, chunk, 0, unroll=False)

  plsc.subcore_barrier()

  @pl.when(s == 0)
  def _():
    pltpu.sync_copy(sums, sums_hbm.at[c])
    pltpu.sync_copy(cnts, cnts_hbm.at[c])


def _pool_sc(h, batch_r, ones, zeros):
  mesh = plsc.VectorSubcoreMesh(core_axis_name="c", subcore_axis_name="s")
  return pl.kernel(
      _pool_body,
      out_type=(jax.ShapeDtypeStruct((NC, G, D), jnp.float32),
                jax.ShapeDtypeStruct((NC, G, D), jnp.float32)),
      mesh=mesh,
      scratch_types=[
          pltpu.VMEM((N_CHUNKS, POOL_CHUNK), jnp.int32),
          pltpu.VMEM((POOL_CHUNK, D), jnp.float32),
          pltpu.VMEM((POOL_CHUNK, D), jnp.float32),
          pltpu.VMEM_SHARED((G, D), jnp.float32),
          pltpu.VMEM_SHARED((G, D), jnp.float32),
          pltpu.SemaphoreType.DMA,
      ],
  )(h, batch_r, ones, zeros)


BN = 1000


def _dense_body(relu, p0_ref, p1_ref, h_ref, wr_ref, wroot_ref, b_ref,
                out_ref):
  agg = (p0_ref[...].astype(jnp.float32) + p1_ref[...].astype(jnp.float32))
  r = (jnp.dot(agg, wr_ref[...], preferred_element_type=jnp.float32)
       + jnp.dot(h_ref[...].astype(jnp.float32), wroot_ref[...],
                 preferred_element_type=jnp.float32)
       + b_ref[...])
  r = jnp.maximum(r, 0.0) if relu else r
  out_ref[...] = r.astype(out_ref.dtype)


def _dense_layer(parts, h, wr, wroot, b, relu, out_dtype):
  p0 = parts[0]
  p1 = parts[1]
  b2 = b.reshape(1, D)
  grid = (N // BN,)
  return pl.pallas_call(
      functools.partial(_dense_body, relu),
      grid=grid,
      in_specs=[
          pl.BlockSpec((BN, D), lambda i: (i, 0)),
          pl.BlockSpec((BN, D), lambda i: (i, 0)),
          pl.BlockSpec((BN, D), lambda i: (i, 0)),
          pl.BlockSpec((D, D), lambda i: (0, 0)),
          pl.BlockSpec((D, D), lambda i: (0, 0)),
          pl.BlockSpec((1, D), lambda i: (0, 0)),
      ],
      out_specs=pl.BlockSpec((BN, D), lambda i: (i, 0)),
      out_shape=jax.ShapeDtypeStruct((N, D), out_dtype),
  )(p0, p1, h, wr, wroot, b2)


def _head_body(s0_ref, s1_ref, c0_ref, c1_ref, wl_ref, bl_ref, out_ref):
  sums = s0_ref[...] + s1_ref[...]
  cnts = c0_ref[...] + c1_ref[...]
  pooled = sums / jnp.maximum(cnts, 1.0)
  out_ref[...] = (jnp.dot(pooled, wl_ref[...],
                          preferred_element_type=jnp.float32)
                  + bl_ref[...])


def _head(sums, cnts, w_lin, b_lin):
  return pl.pallas_call(
      _head_body,
      out_shape=jax.ShapeDtypeStruct((G, 2), jnp.float32),
  )(sums[0], sums[1], cnts[0], cnts[1], w_lin, b_lin.reshape(1, 2))


def kernel(x, edge_index, batch, W1_rel, b1, W1_root, W2_rel, b2, W2_root,
           W3_rel, b3, W3_root, W_lin, b_lin):
  src_r = edge_index[0].reshape(NW, CHUNKS_PER_TILE, EDGE_CHUNK)
  dst_r = edge_index[1].reshape(NW, CHUNKS_PER_TILE, EDGE_CHUNK)
  batch_r = batch.reshape(N_CHUNKS, POOL_CHUNK)
  zeros_n = jnp.zeros((N, D), jnp.bfloat16)
  zeros_g = jnp.zeros((2 * G, D), jnp.float32)
  ones = jnp.ones((POOL_CHUNK, D), jnp.float32)

  h = x.astype(jnp.bfloat16)
  p = _seg_sum_sc(h, src_r, dst_r, zeros_n)
  h = _dense_layer(p, h, W1_rel, W1_root, b1, relu=True,
                   out_dtype=jnp.bfloat16)
  p = _seg_sum_sc(h, src_r, dst_r, zeros_n)
  h = _dense_layer(p, h, W2_rel, W2_root, b2, relu=True,
                   out_dtype=jnp.bfloat16)
  p = _seg_sum_sc(h, src_r, dst_r, zeros_n)
  h = _dense_layer(p, h, W3_rel, W3_root, b3, relu=False,
                   out_dtype=jnp.float32)

  sums, cnts = _pool_sc(h, batch_r, ones, zeros_g)
  return _head(sums, cnts, W_lin, b_lin)


# speedup vs baseline: 12.1858x; 1.0537x over previous
import functools

import jax
import jax.numpy as jnp
from jax import lax
from jax.experimental import pallas as pl
from jax.experimental.pallas import tpu as pltpu
from jax.experimental.pallas import tpu_sc as plsc

N = 10000
E = 320000
D = 128
G = 64

NC = 2
NS = 16
NW = NC * NS

EDGE_CHUNK = 125
KDEPTH = 8
E_PER_TILE = E // NW
CHUNKS_PER_TILE = E_PER_TILE // EDGE_CHUNK
GROUPS_PER_TILE = CHUNKS_PER_TILE // KDEPTH
POOL_CHUNK = 80
N_CHUNKS = N // POOL_CHUNK
ROWS_T = 624
ROWS_TAIL = N - NS * ROWS_T


def _seg_sum_body(h_hbm, srcr_hbm, dstr_hbm, zeros_hbm, out_hbm,
                  src_v, dst_v, rows_bufs, acc, sems):
  c = lax.axis_index("c")
  s = lax.axis_index("s")
  w = c * NS + s

  zbase = pl.multiple_of(s * ROWS_T, 8)
  pltpu.sync_copy(zeros_hbm.at[pl.ds(zbase, ROWS_T)],
                  acc.at[pl.ds(zbase, ROWS_T)])

  @pl.when(s == NS - 1)
  def _():
    pltpu.sync_copy(zeros_hbm.at[pl.ds(NS * ROWS_T, ROWS_TAIL)],
                    acc.at[pl.ds(NS * ROWS_T, ROWS_TAIL)])

  pltpu.sync_copy(srcr_hbm.at[w], src_v)
  pltpu.sync_copy(dstr_hbm.at[w], dst_v)
  plsc.subcore_barrier()

  def group(i, carry):
    g0 = i * KDEPTH
    for b in range(KDEPTH):
      pltpu.async_copy(h_hbm.at[src_v.at[g0 + b]], rows_bufs.at[b], sems[b])
    for b in range(KDEPTH):
      pltpu.make_async_copy(h_hbm.at[src_v.at[g0 + b]], rows_bufs.at[b],
                            sems[b]).wait()
      pltpu.sync_copy(rows_bufs.at[b], acc.at[dst_v.at[g0 + b]], add=True)
    return carry

  lax.fori_loop(0, GROUPS_PER_TILE, group, 0, unroll=False)

  plsc.subcore_barrier()
  pltpu.sync_copy(acc.at[pl.ds(zbase, ROWS_T)],
                  out_hbm.at[c, pl.ds(zbase, ROWS_T)])

  @pl.when(s == NS - 1)
  def _():
    pltpu.sync_copy(acc.at[pl.ds(NS * ROWS_T, ROWS_TAIL)],
                    out_hbm.at[c, pl.ds(NS * ROWS_T, ROWS_TAIL)])


def _seg_sum_sc(h, src_r, dst_r, zeros):
  mesh = plsc.VectorSubcoreMesh(core_axis_name="c", subcore_axis_name="s")
  return pl.kernel(
      _seg_sum_body,
      out_type=jax.ShapeDtypeStruct((NC, N, D), jnp.bfloat16),
      mesh=mesh,
      scratch_types=[
          pltpu.VMEM((CHUNKS_PER_TILE, EDGE_CHUNK), jnp.int32),
          pltpu.VMEM((CHUNKS_PER_TILE, EDGE_CHUNK), jnp.int32),
          pltpu.VMEM((KDEPTH, EDGE_CHUNK, D), jnp.bfloat16),
          pltpu.VMEM_SHARED((N, D), jnp.bfloat16),
          [pltpu.SemaphoreType.DMA] * KDEPTH,
      ],
      compiler_params=pltpu.CompilerParams(use_tc_tiling_on_sc=False),
  )(h, src_r, dst_r, zeros)


def _pool_body(h_hbm, batchr_hbm, ones_hbm, zeros_hbm, sums_hbm, cnts_hbm,
               batch_v, rows_v, ones_v, sums, cnts, gsem):
  c = lax.axis_index("c")
  s = lax.axis_index("s")
  w = c * NS + s

  @pl.when(s == 0)
  def _():
    pltpu.sync_copy(zeros_hbm.at[pl.ds(0, G)], sums)
    pltpu.sync_copy(zeros_hbm.at[pl.ds(G, G)], cnts)

  pltpu.sync_copy(ones_hbm, ones_v)
  pltpu.sync_copy(batchr_hbm, batch_v)
  plsc.subcore_barrier()

  def chunk(k, carry):
    g = w + k * NW
    @pl.when(g < N_CHUNKS)
    def _():
      base = pl.multiple_of(g * POOL_CHUNK, 8)
      pltpu.async_copy(h_hbm.at[pl.ds(base, POOL_CHUNK)], rows_v,
                       gsem).wait()
      pltpu.sync_copy(rows_v, sums.at[batch_v.at[g]], add=True)
      pltpu.sync_copy(ones_v, cnts.at[batch_v.at[g]], add=True)
    return carry

  lax.fori_loop(0, (N_CHUNKS + NW - 1) // NW, chunk, 0, unroll=False)

  plsc.subcore_barrier()

  @pl.when(s == 0)
  def _():
    pltpu.sync_copy(sums, sums_hbm.at[c])
    pltpu.sync_copy(cnts, cnts_hbm.at[c])


def _pool_sc(h, batch_r, ones, zeros):
  mesh = plsc.VectorSubcoreMesh(core_axis_name="c", subcore_axis_name="s")
  return pl.kernel(
      _pool_body,
      out_type=(jax.ShapeDtypeStruct((NC, G, D), jnp.float32),
                jax.ShapeDtypeStruct((NC, G, D), jnp.float32)),
      mesh=mesh,
      scratch_types=[
          pltpu.VMEM((N_CHUNKS, POOL_CHUNK), jnp.int32),
          pltpu.VMEM((POOL_CHUNK, D), jnp.float32),
          pltpu.VMEM((POOL_CHUNK, D), jnp.float32),
          pltpu.VMEM_SHARED((G, D), jnp.float32),
          pltpu.VMEM_SHARED((G, D), jnp.float32),
          pltpu.SemaphoreType.DMA,
      ],
  )(h, batch_r, ones, zeros)


BN = 1000


def _dense_body(relu, p0_ref, p1_ref, h_ref, wr_ref, wroot_ref, b_ref,
                out_ref):
  agg = (p0_ref[...].astype(jnp.float32) + p1_ref[...].astype(jnp.float32))
  r = (jnp.dot(agg, wr_ref[...], preferred_element_type=jnp.float32)
       + jnp.dot(h_ref[...].astype(jnp.float32), wroot_ref[...],
                 preferred_element_type=jnp.float32)
       + b_ref[...])
  r = jnp.maximum(r, 0.0) if relu else r
  out_ref[...] = r.astype(out_ref.dtype)


def _dense_layer(parts, h, wr, wroot, b, relu, out_dtype):
  p0 = parts[0]
  p1 = parts[1]
  b2 = b.reshape(1, D)
  grid = (N // BN,)
  return pl.pallas_call(
      functools.partial(_dense_body, relu),
      grid=grid,
      in_specs=[
          pl.BlockSpec((BN, D), lambda i: (i, 0)),
          pl.BlockSpec((BN, D), lambda i: (i, 0)),
          pl.BlockSpec((BN, D), lambda i: (i, 0)),
          pl.BlockSpec((D, D), lambda i: (0, 0)),
          pl.BlockSpec((D, D), lambda i: (0, 0)),
          pl.BlockSpec((1, D), lambda i: (0, 0)),
      ],
      out_specs=pl.BlockSpec((BN, D), lambda i: (i, 0)),
      out_shape=jax.ShapeDtypeStruct((N, D), out_dtype),
  )(p0, p1, h, wr, wroot, b2)


def _head_body(s0_ref, s1_ref, c0_ref, c1_ref, wl_ref, bl_ref, out_ref):
  sums = s0_ref[...] + s1_ref[...]
  cnts = c0_ref[...] + c1_ref[...]
  pooled = sums / jnp.maximum(cnts, 1.0)
  out_ref[...] = (jnp.dot(pooled, wl_ref[...],
                          preferred_element_type=jnp.float32)
                  + bl_ref[...])


def _head(sums, cnts, w_lin, b_lin):
  return pl.pallas_call(
      _head_body,
      out_shape=jax.ShapeDtypeStruct((G, 2), jnp.float32),
  )(sums[0], sums[1], cnts[0], cnts[1], w_lin, b_lin.reshape(1, 2))


def kernel(x, edge_index, batch, W1_rel, b1, W1_root, W2_rel, b2, W2_root,
           W3_rel, b3, W3_root, W_lin, b_lin):
  src_r = edge_index[0].reshape(NW, CHUNKS_PER_TILE, EDGE_CHUNK)
  dst_r = edge_index[1].reshape(NW, CHUNKS_PER_TILE, EDGE_CHUNK)
  batch_r = batch.reshape(N_CHUNKS, POOL_CHUNK)
  zeros_n = jnp.zeros((N, D), jnp.bfloat16)
  zeros_g = jnp.zeros((2 * G, D), jnp.float32)
  ones = jnp.ones((POOL_CHUNK, D), jnp.float32)

  h = x.astype(jnp.bfloat16)
  p = _seg_sum_sc(h, src_r, dst_r, zeros_n)
  h = _dense_layer(p, h, W1_rel, W1_root, b1, relu=True,
                   out_dtype=jnp.bfloat16)
  p = _seg_sum_sc(h, src_r, dst_r, zeros_n)
  h = _dense_layer(p, h, W2_rel, W2_root, b2, relu=True,
                   out_dtype=jnp.bfloat16)
  p = _seg_sum_sc(h, src_r, dst_r, zeros_n)
  h = _dense_layer(p, h, W3_rel, W3_root, b3, relu=False,
                   out_dtype=jnp.float32)

  sums, cnts = _pool_sc(h, batch_r, ones, zeros_g)
  return _head(sums, cnts, W_lin, b_lin)


# speedup vs baseline: 12.8256x; 1.0525x over previous
import functools

import jax
import jax.numpy as jnp
from jax import lax
from jax.experimental import pallas as pl
from jax.experimental.pallas import tpu as pltpu
from jax.experimental.pallas import tpu_sc as plsc

N = 10000
E = 320000
D = 128
G = 64

NC = 2
NS = 16
NW = NC * NS

EDGE_CHUNK = 125
KDEPTH = 8
E_PER_TILE = E // NW
CHUNKS_PER_TILE = E_PER_TILE // EDGE_CHUNK
GROUPS_PER_TILE = CHUNKS_PER_TILE // KDEPTH
POOL_CHUNK = 80
N_CHUNKS = N // POOL_CHUNK
ROWS_T = 624
ROWS_TAIL = N - NS * ROWS_T


def _seg_sum_body(h_hbm, srcr_hbm, dstr_hbm, zeros_hbm, out_hbm,
                  src_v, dst_v, rows_bufs, acc, sems):
  c = lax.axis_index("c")
  s = lax.axis_index("s")
  w = c * NS + s

  zbase = pl.multiple_of(s * ROWS_T, 8)
  pltpu.sync_copy(zeros_hbm.at[pl.ds(zbase, ROWS_T)],
                  acc.at[pl.ds(zbase, ROWS_T)])

  @pl.when(s == NS - 1)
  def _():
    pltpu.sync_copy(zeros_hbm.at[pl.ds(NS * ROWS_T, ROWS_TAIL)],
                    acc.at[pl.ds(NS * ROWS_T, ROWS_TAIL)])

  pltpu.sync_copy(srcr_hbm.at[w], src_v)
  pltpu.sync_copy(dstr_hbm.at[w], dst_v)
  plsc.subcore_barrier()

  def group(i, carry):
    g0 = i * KDEPTH
    for b in range(KDEPTH):
      pltpu.async_copy(h_hbm.at[src_v.at[g0 + b]], rows_bufs.at[b], sems[b])
    for b in range(KDEPTH):
      pltpu.make_async_copy(h_hbm.at[src_v.at[g0 + b]], rows_bufs.at[b],
                            sems[b]).wait()
      pltpu.sync_copy(rows_bufs.at[b], acc.at[dst_v.at[g0 + b]], add=True)
    return carry

  lax.fori_loop(0, GROUPS_PER_TILE, group, 0, unroll=False)

  plsc.subcore_barrier()
  pltpu.sync_copy(acc.at[pl.ds(zbase, ROWS_T)],
                  out_hbm.at[c, pl.ds(zbase, ROWS_T)])

  @pl.when(s == NS - 1)
  def _():
    pltpu.sync_copy(acc.at[pl.ds(NS * ROWS_T, ROWS_TAIL)],
                    out_hbm.at[c, pl.ds(NS * ROWS_T, ROWS_TAIL)])


def _seg_sum_sc(h, src_r, dst_r, zeros):
  mesh = plsc.VectorSubcoreMesh(core_axis_name="c", subcore_axis_name="s")
  return pl.kernel(
      _seg_sum_body,
      out_type=jax.ShapeDtypeStruct((NC, N, D), jnp.bfloat16),
      mesh=mesh,
      scratch_types=[
          pltpu.VMEM((CHUNKS_PER_TILE, EDGE_CHUNK), jnp.int32),
          pltpu.VMEM((CHUNKS_PER_TILE, EDGE_CHUNK), jnp.int32),
          pltpu.VMEM((KDEPTH, EDGE_CHUNK, D), jnp.bfloat16),
          pltpu.VMEM_SHARED((N, D), jnp.bfloat16),
          [pltpu.SemaphoreType.DMA] * KDEPTH,
      ],
      compiler_params=pltpu.CompilerParams(use_tc_tiling_on_sc=False),
  )(h, src_r, dst_r, zeros)


BN = 1000


def _dense_body(relu, p0_ref, p1_ref, h_ref, wr_ref, wroot_ref, b_ref,
                out_ref):
  agg = (p0_ref[...].astype(jnp.float32) + p1_ref[...].astype(jnp.float32))
  r = (jnp.dot(agg, wr_ref[...], preferred_element_type=jnp.float32)
       + jnp.dot(h_ref[...].astype(jnp.float32), wroot_ref[...],
                 preferred_element_type=jnp.float32)
       + b_ref[...])
  r = jnp.maximum(r, 0.0) if relu else r
  out_ref[...] = r.astype(out_ref.dtype)


def _dense_layer(parts, h, wr, wroot, b, relu, out_dtype):
  p0 = parts[0]
  p1 = parts[1]
  b2 = b.reshape(1, D)
  grid = (N // BN,)
  return pl.pallas_call(
      functools.partial(_dense_body, relu),
      grid=grid,
      in_specs=[
          pl.BlockSpec((BN, D), lambda i: (i, 0)),
          pl.BlockSpec((BN, D), lambda i: (i, 0)),
          pl.BlockSpec((BN, D), lambda i: (i, 0)),
          pl.BlockSpec((D, D), lambda i: (0, 0)),
          pl.BlockSpec((D, D), lambda i: (0, 0)),
          pl.BlockSpec((1, D), lambda i: (0, 0)),
      ],
      out_specs=pl.BlockSpec((BN, D), lambda i: (i, 0)),
      out_shape=jax.ShapeDtypeStruct((N, D), out_dtype),
  )(p0, p1, h, wr, wroot, b2)


def _tail_body(p0_ref, p1_ref, h_ref, wr_ref, wroot_ref, b_ref,
               batch_ref, wl_ref, bl_ref, out_ref, sum_s, cnt_s):
  i = pl.program_id(0)
  agg = (p0_ref[...].astype(jnp.float32) + p1_ref[...].astype(jnp.float32))
  r = (jnp.dot(agg, wr_ref[...], preferred_element_type=jnp.float32)
       + jnp.dot(h_ref[...].astype(jnp.float32), wroot_ref[...],
                 preferred_element_type=jnp.float32)
       + b_ref[...])

  bb = batch_ref[0, 0, :]
  onehot = (bb[None, :] == lax.broadcasted_iota(jnp.int32, (G, BN), 0)
            ).astype(jnp.float32)
  psum = jnp.dot(onehot, r, preferred_element_type=jnp.float32)
  pcnt = jnp.dot(onehot, jnp.ones((BN, D), jnp.float32),
                 preferred_element_type=jnp.float32)

  @pl.when(i == 0)
  def _():
    sum_s[...] = jnp.zeros((G, D), jnp.float32)
    cnt_s[...] = jnp.zeros((G, D), jnp.float32)

  sum_s[...] += psum
  cnt_s[...] += pcnt

  @pl.when(i == pl.num_programs(0) - 1)
  def _():
    pooled = sum_s[...] / jnp.maximum(cnt_s[...], 1.0)
    out_ref[...] = (jnp.dot(pooled, wl_ref[...],
                            preferred_element_type=jnp.float32)
                    + bl_ref[...])


def _tail(parts, h, wr, wroot, b, batch3, w_lin, b_lin):
  return pl.pallas_call(
      _tail_body,
      grid=(N // BN,),
      in_specs=[
          pl.BlockSpec((BN, D), lambda i: (i, 0)),
          pl.BlockSpec((BN, D), lambda i: (i, 0)),
          pl.BlockSpec((BN, D), lambda i: (i, 0)),
          pl.BlockSpec((D, D), lambda i: (0, 0)),
          pl.BlockSpec((D, D), lambda i: (0, 0)),
          pl.BlockSpec((1, D), lambda i: (0, 0)),
          pl.BlockSpec((1, 1, BN), lambda i: (i, 0, 0)),
          pl.BlockSpec((D, 2), lambda i: (0, 0)),
          pl.BlockSpec((1, 2), lambda i: (0, 0)),
      ],
      out_specs=pl.BlockSpec((G, 2), lambda i: (0, 0)),
      out_shape=jax.ShapeDtypeStruct((G, 2), jnp.float32),
      scratch_shapes=[
          pltpu.VMEM((G, D), jnp.float32),
          pltpu.VMEM((G, D), jnp.float32),
      ],
  )(parts[0], parts[1], h, wr, wroot, b.reshape(1, D), batch3,
    w_lin, b_lin.reshape(1, 2))


def kernel(x, edge_index, batch, W1_rel, b1, W1_root, W2_rel, b2, W2_root,
           W3_rel, b3, W3_root, W_lin, b_lin):
  src_r = edge_index[0].reshape(NW, CHUNKS_PER_TILE, EDGE_CHUNK)
  dst_r = edge_index[1].reshape(NW, CHUNKS_PER_TILE, EDGE_CHUNK)
  batch3 = batch.reshape(N // BN, 1, BN)
  zeros_n = jnp.zeros((N, D), jnp.bfloat16)

  h = x.astype(jnp.bfloat16)
  p = _seg_sum_sc(h, src_r, dst_r, zeros_n)
  h = _dense_layer(p, h, W1_rel, W1_root, b1, relu=True,
                   out_dtype=jnp.bfloat16)
  p = _seg_sum_sc(h, src_r, dst_r, zeros_n)
  h = _dense_layer(p, h, W2_rel, W2_root, b2, relu=True,
                   out_dtype=jnp.bfloat16)
  p = _seg_sum_sc(h, src_r, dst_r, zeros_n)
  return _tail(p, h, W3_rel, W3_root, b3, batch3, W_lin, b_lin)


# speedup vs baseline: 14.3457x; 1.1185x over previous
import functools

import jax
import jax.numpy as jnp
from jax import lax
from jax.experimental import pallas as pl
from jax.experimental.pallas import tpu as pltpu
from jax.experimental.pallas import tpu_sc as plsc

N = 10000
E = 320000
D = 128
G = 64

NC = 2
NS = 16
NW = NC * NS

EDGE_CHUNK = 125
KDEPTH = 8
E_PER_TILE = E // NW
CHUNKS_PER_TILE = E_PER_TILE // EDGE_CHUNK
GROUPS_PER_TILE = CHUNKS_PER_TILE // KDEPTH
POOL_CHUNK = 80
N_CHUNKS = N // POOL_CHUNK
ROWS_T = 624
ROWS_TAIL = N - NS * ROWS_T


def _seg_sum_body(h_hbm, srcr_hbm, dstr_hbm, zeros_hbm, out_hbm,
                  src_v, dst_v, rows_bufs, acc, sems, ssems):
  c = lax.axis_index("c")
  s = lax.axis_index("s")
  w = c * NS + s

  zbase = pl.multiple_of(s * ROWS_T, 8)
  pltpu.sync_copy(zeros_hbm.at[pl.ds(zbase, ROWS_T)],
                  acc.at[pl.ds(zbase, ROWS_T)])

  @pl.when(s == NS - 1)
  def _():
    pltpu.sync_copy(zeros_hbm.at[pl.ds(NS * ROWS_T, ROWS_TAIL)],
                    acc.at[pl.ds(NS * ROWS_T, ROWS_TAIL)])

  pltpu.sync_copy(srcr_hbm.at[w], src_v)
  pltpu.sync_copy(dstr_hbm.at[w], dst_v)
  plsc.subcore_barrier()

  for b in range(KDEPTH):
    pltpu.async_copy(h_hbm.at[src_v.at[b]], rows_bufs.at[b], sems[b])

  def group(i, carry):
    g0 = i * KDEPTH
    for b in range(KDEPTH):
      pltpu.make_async_copy(h_hbm.at[src_v.at[g0 + b]], rows_bufs.at[b],
                            sems[b]).wait()
      pltpu.async_copy(rows_bufs.at[b], acc.at[dst_v.at[g0 + b]], ssems[b],
                       add=True)
    for b in range(KDEPTH):
      pltpu.make_async_copy(rows_bufs.at[b], acc.at[dst_v.at[g0 + b]],
                            ssems[b]).wait()

      @pl.when(i + 1 < GROUPS_PER_TILE)
      def _():
        pltpu.async_copy(h_hbm.at[src_v.at[g0 + KDEPTH + b]],
                         rows_bufs.at[b], sems[b])
    return carry

  lax.fori_loop(0, GROUPS_PER_TILE, group, 0, unroll=False)

  plsc.subcore_barrier()
  pltpu.sync_copy(acc.at[pl.ds(zbase, ROWS_T)],
                  out_hbm.at[c, pl.ds(zbase, ROWS_T)])

  @pl.when(s == NS - 1)
  def _():
    pltpu.sync_copy(acc.at[pl.ds(NS * ROWS_T, ROWS_TAIL)],
                    out_hbm.at[c, pl.ds(NS * ROWS_T, ROWS_TAIL)])


def _seg_sum_sc(h, src_r, dst_r, zeros):
  mesh = plsc.VectorSubcoreMesh(core_axis_name="c", subcore_axis_name="s")
  return pl.kernel(
      _seg_sum_body,
      out_type=jax.ShapeDtypeStruct((NC, N, D), jnp.bfloat16),
      mesh=mesh,
      scratch_types=[
          pltpu.VMEM((CHUNKS_PER_TILE, EDGE_CHUNK), jnp.int32),
          pltpu.VMEM((CHUNKS_PER_TILE, EDGE_CHUNK), jnp.int32),
          pltpu.VMEM((KDEPTH, EDGE_CHUNK, D), jnp.bfloat16),
          pltpu.VMEM_SHARED((N, D), jnp.bfloat16),
          [pltpu.SemaphoreType.DMA] * KDEPTH,
          [pltpu.SemaphoreType.DMA] * KDEPTH,
      ],
      compiler_params=pltpu.CompilerParams(use_tc_tiling_on_sc=False),
  )(h, src_r, dst_r, zeros)


BN = 1000


def _dense_body(relu, p0_ref, p1_ref, h_ref, wr_ref, wroot_ref, b_ref,
                out_ref):
  agg = (p0_ref[...].astype(jnp.float32) + p1_ref[...].astype(jnp.float32))
  r = (jnp.dot(agg, wr_ref[...], preferred_element_type=jnp.float32)
       + jnp.dot(h_ref[...].astype(jnp.float32), wroot_ref[...],
                 preferred_element_type=jnp.float32)
       + b_ref[...])
  r = jnp.maximum(r, 0.0) if relu else r
  out_ref[...] = r.astype(out_ref.dtype)


def _dense_layer(parts, h, wr, wroot, b, relu, out_dtype):
  p0 = parts[0]
  p1 = parts[1]
  b2 = b.reshape(1, D)
  grid = (N // BN,)
  return pl.pallas_call(
      functools.partial(_dense_body, relu),
      grid=grid,
      in_specs=[
          pl.BlockSpec((BN, D), lambda i: (i, 0)),
          pl.BlockSpec((BN, D), lambda i: (i, 0)),
          pl.BlockSpec((BN, D), lambda i: (i, 0)),
          pl.BlockSpec((D, D), lambda i: (0, 0)),
          pl.BlockSpec((D, D), lambda i: (0, 0)),
          pl.BlockSpec((1, D), lambda i: (0, 0)),
      ],
      out_specs=pl.BlockSpec((BN, D), lambda i: (i, 0)),
      out_shape=jax.ShapeDtypeStruct((N, D), out_dtype),
  )(p0, p1, h, wr, wroot, b2)


def _tail_body(p0_ref, p1_ref, h_ref, wr_ref, wroot_ref, b_ref,
               batch_ref, wl_ref, bl_ref, out_ref, sum_s, cnt_s):
  i = pl.program_id(0)
  agg = (p0_ref[...].astype(jnp.float32) + p1_ref[...].astype(jnp.float32))
  r = (jnp.dot(agg, wr_ref[...], preferred_element_type=jnp.float32)
       + jnp.dot(h_ref[...].astype(jnp.float32), wroot_ref[...],
                 preferred_element_type=jnp.float32)
       + b_ref[...])

  bb = batch_ref[0, 0, :]
  onehot = (bb[None, :] == lax.broadcasted_iota(jnp.int32, (G, BN), 0)
            ).astype(jnp.float32)
  psum = jnp.dot(onehot, r, preferred_element_type=jnp.float32)
  pcnt = jnp.dot(onehot, jnp.ones((BN, D), jnp.float32),
                 preferred_element_type=jnp.float32)

  @pl.when(i == 0)
  def _():
    sum_s[...] = jnp.zeros((G, D), jnp.float32)
    cnt_s[...] = jnp.zeros((G, D), jnp.float32)

  sum_s[...] += psum
  cnt_s[...] += pcnt

  @pl.when(i == pl.num_programs(0) - 1)
  def _():
    pooled = sum_s[...] / jnp.maximum(cnt_s[...], 1.0)
    out_ref[...] = (jnp.dot(pooled, wl_ref[...],
                            preferred_element_type=jnp.float32)
                    + bl_ref[...])


def _tail(parts, h, wr, wroot, b, batch3, w_lin, b_lin):
  return pl.pallas_call(
      _tail_body,
      grid=(N // BN,),
      in_specs=[
          pl.BlockSpec((BN, D), lambda i: (i, 0)),
          pl.BlockSpec((BN, D), lambda i: (i, 0)),
          pl.BlockSpec((BN, D), lambda i: (i, 0)),
          pl.BlockSpec((D, D), lambda i: (0, 0)),
          pl.BlockSpec((D, D), lambda i: (0, 0)),
          pl.BlockSpec((1, D), lambda i: (0, 0)),
          pl.BlockSpec((1, 1, BN), lambda i: (i, 0, 0)),
          pl.BlockSpec((D, 2), lambda i: (0, 0)),
          pl.BlockSpec((1, 2), lambda i: (0, 0)),
      ],
      out_specs=pl.BlockSpec((G, 2), lambda i: (0, 0)),
      out_shape=jax.ShapeDtypeStruct((G, 2), jnp.float32),
      scratch_shapes=[
          pltpu.VMEM((G, D), jnp.float32),
          pltpu.VMEM((G, D), jnp.float32),
      ],
  )(parts[0], parts[1], h, wr, wroot, b.reshape(1, D), batch3,
    w_lin, b_lin.reshape(1, 2))


def kernel(x, edge_index, batch, W1_rel, b1, W1_root, W2_rel, b2, W2_root,
           W3_rel, b3, W3_root, W_lin, b_lin):
  src_r = edge_index[0].reshape(NW, CHUNKS_PER_TILE, EDGE_CHUNK)
  dst_r = edge_index[1].reshape(NW, CHUNKS_PER_TILE, EDGE_CHUNK)
  batch3 = batch.reshape(N // BN, 1, BN)
  zeros_n = jnp.zeros((N, D), jnp.bfloat16)

  h = x.astype(jnp.bfloat16)
  p = _seg_sum_sc(h, src_r, dst_r, zeros_n)
  h = _dense_layer(p, h, W1_rel, W1_root, b1, relu=True,
                   out_dtype=jnp.bfloat16)
  p = _seg_sum_sc(h, src_r, dst_r, zeros_n)
  h = _dense_layer(p, h, W2_rel, W2_root, b2, relu=True,
                   out_dtype=jnp.bfloat16)
  p = _seg_sum_sc(h, src_r, dst_r, zeros_n)
  return _tail(p, h, W3_rel, W3_root, b3, batch3, W_lin, b_lin)


# speedup vs baseline: 14.6742x; 1.0229x over previous
import functools

import jax
import jax.numpy as jnp
from jax import lax
from jax.experimental import pallas as pl
from jax.experimental.pallas import tpu as pltpu
from jax.experimental.pallas import tpu_sc as plsc

N = 10000
E = 320000
D = 128
G = 64

NC = 2
NS = 16
NW = NC * NS

EDGE_CHUNK = 125
KDEPTH = 8
E_PER_TILE = E // NW
CHUNKS_PER_TILE = E_PER_TILE // EDGE_CHUNK
GROUPS_PER_TILE = CHUNKS_PER_TILE // KDEPTH
POOL_CHUNK = 80
N_CHUNKS = N // POOL_CHUNK
ROWS_T = 624
ROWS_TAIL = N - NS * ROWS_T


def _seg_sum_body(h_hbm, srcr_hbm, dstr_hbm, zeros_hbm, out_hbm,
                  src_v, dst_v, rows_bufs, acc, sems, ssems):
  c = lax.axis_index("c")
  s = lax.axis_index("s")
  w = c * NS + s

  zbase = pl.multiple_of(s * ROWS_T, 8)
  pltpu.async_copy(zeros_hbm.at[pl.ds(zbase, ROWS_T)],
                   acc.at[pl.ds(zbase, ROWS_T)], sems[0])
  pltpu.async_copy(srcr_hbm.at[w], src_v, sems[1])
  pltpu.async_copy(dstr_hbm.at[w], dst_v, sems[2])

  @pl.when(s == NS - 1)
  def _():
    pltpu.async_copy(zeros_hbm.at[pl.ds(NS * ROWS_T, ROWS_TAIL)],
                     acc.at[pl.ds(NS * ROWS_T, ROWS_TAIL)], sems[3]).wait()

  pltpu.make_async_copy(zeros_hbm.at[pl.ds(zbase, ROWS_T)],
                        acc.at[pl.ds(zbase, ROWS_T)], sems[0]).wait()
  pltpu.make_async_copy(srcr_hbm.at[w], src_v, sems[1]).wait()
  pltpu.make_async_copy(dstr_hbm.at[w], dst_v, sems[2]).wait()
  plsc.subcore_barrier()

  for b in range(KDEPTH):
    pltpu.async_copy(h_hbm.at[src_v.at[b]], rows_bufs.at[b], sems[b])

  def group(i, carry):
    g0 = i * KDEPTH
    for b in range(KDEPTH):
      pltpu.make_async_copy(h_hbm.at[src_v.at[g0 + b]], rows_bufs.at[b],
                            sems[b]).wait()
      pltpu.async_copy(rows_bufs.at[b], acc.at[dst_v.at[g0 + b]], ssems[b],
                       add=True)
    for b in range(KDEPTH):
      pltpu.make_async_copy(rows_bufs.at[b], acc.at[dst_v.at[g0 + b]],
                            ssems[b]).wait()

      @pl.when(i + 1 < GROUPS_PER_TILE)
      def _():
        pltpu.async_copy(h_hbm.at[src_v.at[g0 + KDEPTH + b]],
                         rows_bufs.at[b], sems[b])
    return carry

  lax.fori_loop(0, GROUPS_PER_TILE, group, 0, unroll=False)

  plsc.subcore_barrier()
  pltpu.sync_copy(acc.at[pl.ds(zbase, ROWS_T)],
                  out_hbm.at[c, pl.ds(zbase, ROWS_T)])

  @pl.when(s == NS - 1)
  def _():
    pltpu.sync_copy(acc.at[pl.ds(NS * ROWS_T, ROWS_TAIL)],
                    out_hbm.at[c, pl.ds(NS * ROWS_T, ROWS_TAIL)])


def _seg_sum_sc(h, src_r, dst_r, zeros):
  mesh = plsc.VectorSubcoreMesh(core_axis_name="c", subcore_axis_name="s")
  return pl.kernel(
      _seg_sum_body,
      out_type=jax.ShapeDtypeStruct((NC, N, D), jnp.bfloat16),
      mesh=mesh,
      scratch_types=[
          pltpu.VMEM((CHUNKS_PER_TILE, EDGE_CHUNK), jnp.int32),
          pltpu.VMEM((CHUNKS_PER_TILE, EDGE_CHUNK), jnp.int32),
          pltpu.VMEM((KDEPTH, EDGE_CHUNK, D), jnp.bfloat16),
          pltpu.VMEM_SHARED((N, D), jnp.bfloat16),
          [pltpu.SemaphoreType.DMA] * KDEPTH,
          [pltpu.SemaphoreType.DMA] * KDEPTH,
      ],
      compiler_params=pltpu.CompilerParams(use_tc_tiling_on_sc=False),
  )(h, src_r, dst_r, zeros)


BN = 1000


def _dense_body(relu, p0_ref, p1_ref, h_ref, wr_ref, wroot_ref, b_ref,
                out_ref):
  agg = (p0_ref[...].astype(jnp.float32) + p1_ref[...].astype(jnp.float32))
  r = (jnp.dot(agg, wr_ref[...], preferred_element_type=jnp.float32)
       + jnp.dot(h_ref[...].astype(jnp.float32), wroot_ref[...],
                 preferred_element_type=jnp.float32)
       + b_ref[...])
  r = jnp.maximum(r, 0.0) if relu else r
  out_ref[...] = r.astype(out_ref.dtype)


def _dense_layer(parts, h, wr, wroot, b, relu, out_dtype):
  p0 = parts[0]
  p1 = parts[1]
  b2 = b.reshape(1, D)
  grid = (N // BN,)
  return pl.pallas_call(
      functools.partial(_dense_body, relu),
      grid=grid,
      in_specs=[
          pl.BlockSpec((BN, D), lambda i: (i, 0)),
          pl.BlockSpec((BN, D), lambda i: (i, 0)),
          pl.BlockSpec((BN, D), lambda i: (i, 0)),
          pl.BlockSpec((D, D), lambda i: (0, 0)),
          pl.BlockSpec((D, D), lambda i: (0, 0)),
          pl.BlockSpec((1, D), lambda i: (0, 0)),
      ],
      out_specs=pl.BlockSpec((BN, D), lambda i: (i, 0)),
      out_shape=jax.ShapeDtypeStruct((N, D), out_dtype),
  )(p0, p1, h, wr, wroot, b2)


def _tail_body(p0_ref, p1_ref, h_ref, wr_ref, wroot_ref, b_ref,
               batch_ref, wl_ref, bl_ref, out_ref, sum_s, cnt_s):
  i = pl.program_id(0)
  agg = (p0_ref[...].astype(jnp.float32) + p1_ref[...].astype(jnp.float32))
  r = (jnp.dot(agg, wr_ref[...], preferred_element_type=jnp.float32)
       + jnp.dot(h_ref[...].astype(jnp.float32), wroot_ref[...],
                 preferred_element_type=jnp.float32)
       + b_ref[...])

  bb = batch_ref[0, 0, :]
  onehot = (bb[None, :] == lax.broadcasted_iota(jnp.int32, (G, BN), 0)
            ).astype(jnp.float32)
  psum = jnp.dot(onehot, r, preferred_element_type=jnp.float32)
  pcnt = jnp.dot(onehot, jnp.ones((BN, D), jnp.float32),
                 preferred_element_type=jnp.float32)

  @pl.when(i == 0)
  def _():
    sum_s[...] = jnp.zeros((G, D), jnp.float32)
    cnt_s[...] = jnp.zeros((G, D), jnp.float32)

  sum_s[...] += psum
  cnt_s[...] += pcnt

  @pl.when(i == pl.num_programs(0) - 1)
  def _():
    pooled = sum_s[...] / jnp.maximum(cnt_s[...], 1.0)
    out_ref[...] = (jnp.dot(pooled, wl_ref[...],
                            preferred_element_type=jnp.float32)
                    + bl_ref[...])


def _tail(parts, h, wr, wroot, b, batch3, w_lin, b_lin):
  return pl.pallas_call(
      _tail_body,
      grid=(N // BN,),
      in_specs=[
          pl.BlockSpec((BN, D), lambda i: (i, 0)),
          pl.BlockSpec((BN, D), lambda i: (i, 0)),
          pl.BlockSpec((BN, D), lambda i: (i, 0)),
          pl.BlockSpec((D, D), lambda i: (0, 0)),
          pl.BlockSpec((D, D), lambda i: (0, 0)),
          pl.BlockSpec((1, D), lambda i: (0, 0)),
          pl.BlockSpec((1, 1, BN), lambda i: (i, 0, 0)),
          pl.BlockSpec((D, 2), lambda i: (0, 0)),
          pl.BlockSpec((1, 2), lambda i: (0, 0)),
      ],
      out_specs=pl.BlockSpec((G, 2), lambda i: (0, 0)),
      out_shape=jax.ShapeDtypeStruct((G, 2), jnp.float32),
      scratch_shapes=[
          pltpu.VMEM((G, D), jnp.float32),
          pltpu.VMEM((G, D), jnp.float32),
      ],
  )(parts[0], parts[1], h, wr, wroot, b.reshape(1, D), batch3,
    w_lin, b_lin.reshape(1, 2))


def kernel(x, edge_index, batch, W1_rel, b1, W1_root, W2_rel, b2, W2_root,
           W3_rel, b3, W3_root, W_lin, b_lin):
  src_r = edge_index[0].reshape(NW, CHUNKS_PER_TILE, EDGE_CHUNK)
  dst_r = edge_index[1].reshape(NW, CHUNKS_PER_TILE, EDGE_CHUNK)
  batch3 = batch.reshape(N // BN, 1, BN)
  zeros_n = jnp.zeros((N, D), jnp.bfloat16)

  h = x.astype(jnp.bfloat16)
  p = _seg_sum_sc(h, src_r, dst_r, zeros_n)
  h = _dense_layer(p, h, W1_rel, W1_root, b1, relu=True,
                   out_dtype=jnp.bfloat16)
  p = _seg_sum_sc(h, src_r, dst_r, zeros_n)
  h = _dense_layer(p, h, W2_rel, W2_root, b2, relu=True,
                   out_dtype=jnp.bfloat16)
  p = _seg_sum_sc(h, src_r, dst_r, zeros_n)
  return _tail(p, h, W3_rel, W3_root, b3, batch3, W_lin, b_lin)
